# baseline (speedup 1.0000x reference)
import jax
import jax.numpy as jnp
from jax import lax
from jax.experimental import pallas as pl

N = 10000
E = 320000
D_IN = 128
H = 64
NHID = 128
OUT = 10
NUM_LAYERS = 10
EPS = 0.1

_BLK = 1000


def _dot_bias_add(inp, wT, u, b):
    d_in, d_out = wT.shape

    def body(x_ref, w_ref, u_ref, b_ref, o_ref):
        o_ref[...] = (
            jnp.dot(x_ref[...], w_ref[...],
                    preferred_element_type=jnp.float32)
            + u_ref[...] + b_ref[...])

    return pl.pallas_call(
        body,
        grid=(N // _BLK,),
        in_specs=[
            pl.BlockSpec((_BLK, d_in), lambda i: (i, 0)),
            pl.BlockSpec((d_in, d_out), lambda i: (0, 0)),
            pl.BlockSpec((_BLK, d_out), lambda i: (i, 0)),
            pl.BlockSpec((1, d_out), lambda i: (0, 0)),
        ],
        out_specs=pl.BlockSpec((_BLK, d_out), lambda i: (i, 0)),
        out_shape=jax.ShapeDtypeStruct((N, d_out), jnp.float32),
    )(inp, wT, u, b)


def _dot(inp, wT):
    d_in, d_out = wT.shape

    def body(x_ref, w_ref, o_ref):
        o_ref[...] = jnp.dot(x_ref[...], w_ref[...],
                             preferred_element_type=jnp.float32)

    return pl.pallas_call(
        body,
        grid=(N // _BLK,),
        in_specs=[
            pl.BlockSpec((_BLK, d_in), lambda i: (i, 0)),
            pl.BlockSpec((d_in, d_out), lambda i: (0, 0)),
        ],
        out_specs=pl.BlockSpec((_BLK, d_out), lambda i: (i, 0)),
        out_shape=jax.ShapeDtypeStruct((N, d_out), jnp.float32),
    )(inp, wT)


def _update_project(state, a, v, w, vmat, scale, vnextT):
    def body(s_ref, a_ref, v_ref, w_ref, vm_ref, vn_ref, o_ref, y_ref):
        upd = s_ref[...] + scale * (
            jnp.dot(a_ref[...], w_ref[...],
                    preferred_element_type=jnp.float32)
            + jnp.dot(v_ref[...], vm_ref[...],
                      preferred_element_type=jnp.float32))
        o_ref[...] = upd
        y_ref[...] = jnp.dot(upd, vn_ref[...],
                             preferred_element_type=jnp.float32)

    return pl.pallas_call(
        body,
        grid=(N // _BLK,),
        in_specs=[
            pl.BlockSpec((_BLK, H), lambda i: (i, 0)),
            pl.BlockSpec((_BLK, H), lambda i: (i, 0)),
            pl.BlockSpec((_BLK, H), lambda i: (i, 0)),
            pl.BlockSpec((H, H), lambda i: (0, 0)),
            pl.BlockSpec((H, H), lambda i: (0, 0)),
            pl.BlockSpec((H, H), lambda i: (0, 0)),
        ],
        out_specs=(pl.BlockSpec((_BLK, H), lambda i: (i, 0)),
                   pl.BlockSpec((_BLK, H), lambda i: (i, 0))),
        out_shape=(jax.ShapeDtypeStruct((N, H), jnp.float32),
                   jax.ShapeDtypeStruct((N, H), jnp.float32)),
    )(state, a, v, w, vmat, vnextT)


def _gelu(v):
    return 0.5 * v * (1.0 + lax.erf(v * (2.0 ** -0.5)))


def _readout(h, gamma, beta, w1T, b1, w2T, b2, w3T, b3):
    def body(h_ref, g_ref, be_ref, w1_ref, b1_ref, w2_ref, b2_ref,
             w3_ref, b3_ref, o_ref):
        hh = h_ref[...]
        mu = jnp.mean(hh, axis=1, keepdims=True)
        var = jnp.mean((hh - mu) ** 2, axis=1, keepdims=True)
        hn = (hh - mu) / jnp.sqrt(var + 1e-5) * g_ref[...] + be_ref[...]
        h1 = _gelu(jnp.dot(hn, w1_ref[...],
                           preferred_element_type=jnp.float32) + b1_ref[...])
        h2 = _gelu(jnp.dot(h1, w2_ref[...],
                           preferred_element_type=jnp.float32) + b2_ref[...])
        o_ref[...] = jnp.dot(h2, w3_ref[...],
                             preferred_element_type=jnp.float32) + b3_ref[...]

    return pl.pallas_call(
        body,
        grid=(N // _BLK,),
        in_specs=[
            pl.BlockSpec((_BLK, NHID), lambda i: (i, 0)),
            pl.BlockSpec((1, NHID), lambda i: (0, 0)),
            pl.BlockSpec((1, NHID), lambda i: (0, 0)),
            pl.BlockSpec((NHID, NHID // 2), lambda i: (0, 0)),
            pl.BlockSpec((1, NHID // 2), lambda i: (0, 0)),
            pl.BlockSpec((NHID // 2, NHID // 2), lambda i: (0, 0)),
            pl.BlockSpec((1, NHID // 2), lambda i: (0, 0)),
            pl.BlockSpec((NHID // 2, OUT), lambda i: (0, 0)),
            pl.BlockSpec((1, OUT), lambda i: (0, 0)),
        ],
        out_specs=pl.BlockSpec((_BLK, OUT), lambda i: (i, 0)),
        out_shape=jax.ShapeDtypeStruct((N, OUT), jnp.float32),
    )(h, gamma, beta, w1T, b1, w2T, b2, w3T, b3)


def kernel(x, edge_index, W_emb, b_emb, Wp, Vp, bp, Wq, Vq, bq,
           gamma, beta, W1, b1, W2, b2, W3, b3):
    src, dst = edge_index[0], edge_index[1]

    def agg(y):
        return jnp.zeros((N, H), jnp.float32).at[dst].add(y[src])

    def agg_t(a):
        return jnp.zeros((N, H), jnp.float32).at[src].add(a[dst])

    WpT, VpT = Wp.T, Vp.T
    WqT, VqT = Wq.T, Vq.T
    bp2, bq2 = bp[None], bq[None]

    h = jax.nn.gelu(
        _dot_bias_add(x, W_emb.T, jnp.zeros((N, H), jnp.float32),
                      b_emb[None]),
        approximate=False)
    p = q = h
    yq = _dot(q, VqT)
    for _ in range(NUM_LAYERS):
        aq = jnp.tanh(_dot_bias_add(q, WqT, agg(yq), bq2))
        p, yp = _update_project(p, aq, agg_t(aq), Wq, Vq, -EPS, VpT)
        ap = jnp.tanh(_dot_bias_add(p, WpT, agg(yp), bp2))
        q, yq = _update_project(q, ap, agg_t(ap), Wp, Vp, EPS, VqT)

    h2 = jnp.concatenate([p, q], axis=1)
    return _readout(h2, gamma[None], beta[None], W1.T, b1[None],
                    W2.T, b2[None], W3.T, b3[None])


# speedup vs baseline: 1.0383x; 1.0383x over previous
import jax
import jax.numpy as jnp
from jax import lax
from jax.experimental import pallas as pl

N = 10000
E = 320000
D_IN = 128
H = 64
NHID = 128
OUT = 10
NUM_LAYERS = 10
EPS = 0.1

_BLK = 1000


def _dot_bias_add(inp, wT, u, b, tanh=False):
    d_in, d_out = wT.shape

    def body(x_ref, w_ref, u_ref, b_ref, o_ref):
        z = (jnp.dot(x_ref[...], w_ref[...],
                     preferred_element_type=jnp.float32)
             + u_ref[...] + b_ref[...])
        o_ref[...] = jnp.tanh(z) if tanh else z

    return pl.pallas_call(
        body,
        grid=(N // _BLK,),
        in_specs=[
            pl.BlockSpec((_BLK, d_in), lambda i: (i, 0)),
            pl.BlockSpec((d_in, d_out), lambda i: (0, 0)),
            pl.BlockSpec((_BLK, d_out), lambda i: (i, 0)),
            pl.BlockSpec((1, d_out), lambda i: (0, 0)),
        ],
        out_specs=pl.BlockSpec((_BLK, d_out), lambda i: (i, 0)),
        out_shape=jax.ShapeDtypeStruct((N, d_out), jnp.float32),
    )(inp, wT, u, b)


def _dot(inp, wT):
    d_in, d_out = wT.shape

    def body(x_ref, w_ref, o_ref):
        o_ref[...] = jnp.dot(x_ref[...], w_ref[...],
                             preferred_element_type=jnp.float32)

    return pl.pallas_call(
        body,
        grid=(N // _BLK,),
        in_specs=[
            pl.BlockSpec((_BLK, d_in), lambda i: (i, 0)),
            pl.BlockSpec((d_in, d_out), lambda i: (0, 0)),
        ],
        out_specs=pl.BlockSpec((_BLK, d_out), lambda i: (i, 0)),
        out_shape=jax.ShapeDtypeStruct((N, d_out), jnp.float32),
    )(inp, wT)


def _update_project(state, a, v, w, vmat, scale, vnextT):
    def body(s_ref, a_ref, v_ref, w_ref, vm_ref, vn_ref, o_ref, y_ref):
        upd = s_ref[...] + scale * (
            jnp.dot(a_ref[...], w_ref[...],
                    preferred_element_type=jnp.float32)
            + jnp.dot(v_ref[...], vm_ref[...],
                      preferred_element_type=jnp.float32))
        o_ref[...] = upd
        y_ref[...] = jnp.dot(upd, vn_ref[...],
                             preferred_element_type=jnp.float32)

    return pl.pallas_call(
        body,
        grid=(N // _BLK,),
        in_specs=[
            pl.BlockSpec((_BLK, H), lambda i: (i, 0)),
            pl.BlockSpec((_BLK, H), lambda i: (i, 0)),
            pl.BlockSpec((_BLK, H), lambda i: (i, 0)),
            pl.BlockSpec((H, H), lambda i: (0, 0)),
            pl.BlockSpec((H, H), lambda i: (0, 0)),
            pl.BlockSpec((H, H), lambda i: (0, 0)),
        ],
        out_specs=(pl.BlockSpec((_BLK, H), lambda i: (i, 0)),
                   pl.BlockSpec((_BLK, H), lambda i: (i, 0))),
        out_shape=(jax.ShapeDtypeStruct((N, H), jnp.float32),
                   jax.ShapeDtypeStruct((N, H), jnp.float32)),
    )(state, a, v, w, vmat, vnextT)


def _gelu(v):
    return 0.5 * v * (1.0 + lax.erf(v * (2.0 ** -0.5)))


def _readout(h, gamma, beta, w1T, b1, w2T, b2, w3T, b3):
    def body(h_ref, g_ref, be_ref, w1_ref, b1_ref, w2_ref, b2_ref,
             w3_ref, b3_ref, o_ref):
        hh = h_ref[...]
        mu = jnp.mean(hh, axis=1, keepdims=True)
        var = jnp.mean((hh - mu) ** 2, axis=1, keepdims=True)
        hn = (hh - mu) / jnp.sqrt(var + 1e-5) * g_ref[...] + be_ref[...]
        h1 = _gelu(jnp.dot(hn, w1_ref[...],
                           preferred_element_type=jnp.float32) + b1_ref[...])
        h2 = _gelu(jnp.dot(h1, w2_ref[...],
                           preferred_element_type=jnp.float32) + b2_ref[...])
        o_ref[...] = jnp.dot(h2, w3_ref[...],
                             preferred_element_type=jnp.float32) + b3_ref[...]

    return pl.pallas_call(
        body,
        grid=(N // _BLK,),
        in_specs=[
            pl.BlockSpec((_BLK, NHID), lambda i: (i, 0)),
            pl.BlockSpec((1, NHID), lambda i: (0, 0)),
            pl.BlockSpec((1, NHID), lambda i: (0, 0)),
            pl.BlockSpec((NHID, NHID // 2), lambda i: (0, 0)),
            pl.BlockSpec((1, NHID // 2), lambda i: (0, 0)),
            pl.BlockSpec((NHID // 2, NHID // 2), lambda i: (0, 0)),
            pl.BlockSpec((1, NHID // 2), lambda i: (0, 0)),
            pl.BlockSpec((NHID // 2, OUT), lambda i: (0, 0)),
            pl.BlockSpec((1, OUT), lambda i: (0, 0)),
        ],
        out_specs=pl.BlockSpec((_BLK, OUT), lambda i: (i, 0)),
        out_shape=jax.ShapeDtypeStruct((N, OUT), jnp.float32),
    )(h, gamma, beta, w1T, b1, w2T, b2, w3T, b3)


def kernel(x, edge_index, W_emb, b_emb, Wp, Vp, bp, Wq, Vq, bq,
           gamma, beta, W1, b1, W2, b2, W3, b3):
    src, dst = edge_index[0], edge_index[1]

    def agg(y):
        return jnp.zeros((N, H), jnp.float32).at[dst].add(y[src])

    def agg_t(a):
        return jnp.zeros((N, H), jnp.float32).at[src].add(a[dst])

    WpT, VpT = Wp.T, Vp.T
    WqT, VqT = Wq.T, Vq.T
    bp2, bq2 = bp[None], bq[None]

    h = jax.nn.gelu(
        _dot_bias_add(x, W_emb.T, jnp.zeros((N, H), jnp.float32),
                      b_emb[None]),
        approximate=False)
    p = q = h
    yq = _dot(q, VqT)
    for _ in range(NUM_LAYERS):
        aq = _dot_bias_add(q, WqT, agg(yq), bq2, tanh=True)
        p, yp = _update_project(p, aq, agg_t(aq), Wq, Vq, -EPS, VpT)
        ap = _dot_bias_add(p, WpT, agg(yp), bp2, tanh=True)
        q, yq = _update_project(q, ap, agg_t(ap), Wp, Vp, EPS, VqT)

    h2 = jnp.concatenate([p, q], axis=1)
    return _readout(h2, gamma[None], beta[None], W1.T, b1[None],
                    W2.T, b2[None], W3.T, b3[None])
